# baseline (device time: 82047 ns/iter reference)
import functools

import jax
import jax.numpy as jnp
from jax import lax
from jax.experimental import pallas as pl
from jax.experimental.pallas import tpu as pltpu

N_DEV = 4
HQ = 8
DH = 128
SQ = 256
SKV = 4096
NR = 4
SKR = SKV // NR
D = HQ * DH
SCALE = 0.08838834764831843

_MESH = pl.DeviceIdType.MESH
_PREC = lax.Precision.DEFAULT


def kernel(x, Wq, K_ext, V_ext, Wo):
    x2 = x[0]
    K = K_ext[0].reshape(16, NR, 64, HQ, DH)
    V = V_ext[0].reshape(16, NR, 64, HQ, DH)

    def body(x_ref, wq_ref, k_ref, v_ref, wo_ref, out_ref,
             xbf, xgat, qscr, kstage, vstage, pacc, pstat,
             rbuf, rstat, xsend, xrecv, asend, arecv, ssend, srecv, cpy,
             kcpy, vcpy):
        my = lax.axis_index("i")

        bar = pltpu.get_barrier_semaphore()
        for j in range(1, N_DEV):
            pl.semaphore_signal(bar, inc=1, device_id=((my + j) % N_DEV,),
                                device_id_type=_MESH)
        pl.semaphore_wait(bar, N_DEV - 1)

        pending = []

        def stage_start(h, slot):
            pltpu.make_async_copy(k_ref.at[:, :, :, h, :], kstage.at[slot],
                                  kcpy.at[slot]).start()
            pltpu.make_async_copy(v_ref.at[:, :, :, h, :], vstage.at[slot],
                                  vcpy.at[slot]).start()

        def stage_wait(h, slot):
            pltpu.make_async_copy(k_ref.at[:, :, :, h, :], kstage.at[slot],
                                  kcpy.at[slot]).wait()
            pltpu.make_async_copy(v_ref.at[:, :, :, h, :], vstage.at[slot],
                                  vcpy.at[slot]).wait()

        stage_start(0, 0)
        stage_start(1, 1)

        xbf[...] = x_ref[...].astype(jnp.bfloat16)
        for j in range(1, N_DEV):
            dst = (my + j) % N_DEV
            rx = pltpu.make_async_remote_copy(
                xbf, xgat.at[my], xsend.at[j - 1], xrecv.at[my],
                device_id=(dst,), device_id_type=_MESH)
            rx.start()
            pending.append(rx)
        cx = pltpu.make_async_copy(xbf, xgat.at[my], cpy.at[0])
        cx.start()
        cx.wait()
        for j in range(1, N_DEV):
            src = (my + j) % N_DEV
            pltpu.make_async_remote_copy(
                xbf, xgat.at[src], xsend.at[j - 1], xrecv.at[src],
                device_id=(src,), device_id_type=_MESH).wait_recv()

        xall = xgat[...].astype(jnp.float32).reshape(D, D)
        for h in range(HQ):
            qh = lax.dot(xall, wq_ref[:, h * DH:(h + 1) * DH],
                         precision=_PREC,
                         preferred_element_type=jnp.float32)
            q4 = qh.reshape(N_DEV, NR, 64, DH)
            for r in range(NR):
                qscr[h, r] = q4[:, r].reshape(SQ, DH)

        def head_body(h, carry):
            slot = h % 3
            stage_wait(h, slot)
            stage_start(jnp.minimum(h + 2, HQ - 1), (h + 2) % 3)
            for r in range(NR):
                qr = qscr[h, r]
                kt = kstage[slot, :, r].reshape(SKR, DH)
                vt = vstage[slot, :, r].reshape(SKR, DH)
                s = lax.dot_general(
                    qr, kt, (((1,), (1,)), ((), ())),
                    precision=_PREC,
                    preferred_element_type=jnp.float32) * SCALE
                m_r = jnp.max(s, axis=1, keepdims=True)
                p = jnp.exp(s - m_r)
                l_r = jnp.sum(p, axis=1, keepdims=True)
                a_r = lax.dot(p, vt,
                              precision=_PREC,
                              preferred_element_type=jnp.float32)
                pacc[:, h, r] = a_r.astype(jnp.bfloat16).reshape(
                    N_DEV, 64, DH)
                pstat[:, h, r] = jnp.concatenate(
                    [m_r, l_r], axis=1).reshape(N_DEV, 64, 2)
            return carry

        HH = HQ // 2

        def half_send(half):
            lo = half * HH
            for j in range(1, N_DEV):
                dst = (my + j) % N_DEV
                ra = pltpu.make_async_remote_copy(
                    pacc.at[dst].at[pl.ds(lo, HH)],
                    rbuf.at[my].at[pl.ds(lo, HH)],
                    asend.at[half].at[j - 1], arecv.at[half].at[my],
                    device_id=(dst,), device_id_type=_MESH)
                rs = pltpu.make_async_remote_copy(
                    pstat.at[dst].at[pl.ds(lo, HH)],
                    rstat.at[my].at[pl.ds(lo, HH)],
                    ssend.at[half].at[j - 1], srecv.at[half].at[my],
                    device_id=(dst,), device_id_type=_MESH)
                ra.start()
                rs.start()
                pending.append(ra)
                pending.append(rs)
            ca = pltpu.make_async_copy(
                pacc.at[my].at[pl.ds(lo, HH)],
                rbuf.at[my].at[pl.ds(lo, HH)], cpy.at[2 * half + 1])
            cs = pltpu.make_async_copy(
                pstat.at[my].at[pl.ds(lo, HH)],
                rstat.at[my].at[pl.ds(lo, HH)], cpy.at[2 * half + 2])
            ca.start()
            cs.start()
            return ca, cs

        def half_wait(half, own):
            own[0].wait()
            own[1].wait()
            lo = half * HH
            for j in range(1, N_DEV):
                src = (my + j) % N_DEV
                pltpu.make_async_remote_copy(
                    pacc.at[src].at[pl.ds(lo, HH)],
                    rbuf.at[src].at[pl.ds(lo, HH)],
                    asend.at[half].at[j - 1], arecv.at[half].at[src],
                    device_id=(src,), device_id_type=_MESH).wait_recv()
                pltpu.make_async_remote_copy(
                    pstat.at[src].at[pl.ds(lo, HH)],
                    rstat.at[src].at[pl.ds(lo, HH)],
                    ssend.at[half].at[j - 1], srecv.at[half].at[src],
                    device_id=(src,), device_id_type=_MESH).wait_recv()

        def combine_half(half):
            lo = half * HH
            a = rbuf[:, lo:lo + HH].astype(jnp.float32).reshape(
                N_DEV, HH * SQ, DH)
            st = rstat[:, lo:lo + HH].reshape(N_DEV, HH * SQ, 2)
            m = st[:, :, 0:1]
            l = st[:, :, 1:2]
            mg = jnp.max(m, axis=0, keepdims=True)
            w = jnp.exp(m - mg)
            num = jnp.sum(w * a, axis=0)
            den = jnp.sum(w * l, axis=0)
            ctx = (num / den).reshape(HH, SQ, DH).transpose(1, 0, 2
                                                           ).reshape(SQ,
                                                                     HH * DH)
            wo_h = wo_ref[lo * DH:(lo + HH) * DH, :]
            return lax.dot(ctx, wo_h, precision=_PREC,
                           preferred_element_type=jnp.float32)

        lax.fori_loop(0, HQ, head_body, 0)
        stage_wait(HQ - 1, 2)
        stage_wait(HQ - 1, 0)
        own0 = half_send(0)
        own1 = half_send(1)
        half_wait(0, own0)
        out0 = combine_half(0)
        half_wait(1, own1)
        out_ref[...] = out0 + combine_half(1)

        for d in pending:
            d.wait_send()

        @functools.partial(pl.run_scoped, sem2=pltpu.SemaphoreType.REGULAR)
        def _(sem2):
            for j in range(1, N_DEV):
                pl.semaphore_signal(sem2, inc=1,
                                    device_id=((my + j) % N_DEV,),
                                    device_id_type=_MESH)
            pl.semaphore_wait(sem2, N_DEV - 1)

    out = pl.pallas_call(
        body,
        out_shape=jax.ShapeDtypeStruct((SQ, D), jnp.float32),
        in_specs=[
            pl.BlockSpec(memory_space=pltpu.MemorySpace.VMEM),
            pl.BlockSpec(memory_space=pltpu.MemorySpace.VMEM),
            pl.BlockSpec(memory_space=pltpu.MemorySpace.HBM),
            pl.BlockSpec(memory_space=pltpu.MemorySpace.HBM),
            pl.BlockSpec(memory_space=pltpu.MemorySpace.VMEM),
        ],
        out_specs=pl.BlockSpec(memory_space=pltpu.MemorySpace.VMEM),
        scratch_shapes=[
            pltpu.VMEM((SQ, D), jnp.bfloat16),
            pltpu.VMEM((N_DEV, SQ, D), jnp.bfloat16),
            pltpu.VMEM((HQ, NR, SQ, DH), jnp.float32),
            pltpu.VMEM((3, 16, NR, 64, DH), jnp.float32),
            pltpu.VMEM((3, 16, NR, 64, DH), jnp.float32),
            pltpu.VMEM((N_DEV, HQ, NR, 64, DH), jnp.bfloat16),
            pltpu.VMEM((N_DEV, HQ, NR, 64, 2), jnp.float32),
            pltpu.VMEM((N_DEV, HQ, NR, 64, DH), jnp.bfloat16),
            pltpu.VMEM((N_DEV, HQ, NR, 64, 2), jnp.float32),
            pltpu.SemaphoreType.DMA((N_DEV - 1,)),
            pltpu.SemaphoreType.DMA((N_DEV,)),
            pltpu.SemaphoreType.DMA((2, N_DEV - 1)),
            pltpu.SemaphoreType.DMA((2, N_DEV)),
            pltpu.SemaphoreType.DMA((2, N_DEV - 1)),
            pltpu.SemaphoreType.DMA((2, N_DEV)),
            pltpu.SemaphoreType.DMA((5,)),
            pltpu.SemaphoreType.DMA((3,)),
            pltpu.SemaphoreType.DMA((3,)),
        ],
        compiler_params=pltpu.CompilerParams(
            collective_id=0, vmem_limit_bytes=52 * 1024 * 1024),
    )(x2, Wq, K, V, Wo)
    return out[None]


# device time: 73663 ns/iter; 1.1138x vs baseline; 1.1138x over previous
import functools

import jax
import jax.numpy as jnp
from jax import lax
from jax.experimental import pallas as pl
from jax.experimental.pallas import tpu as pltpu

N_DEV = 4
HQ = 8
DH = 128
SQ = 256
SKV = 4096
NR = 4
SKR = SKV // NR
D = HQ * DH
SCALE = 0.08838834764831843

_MESH = pl.DeviceIdType.MESH
_PREC = lax.Precision.DEFAULT


def kernel(x, Wq, K_ext, V_ext, Wo):
    x2 = x[0]
    K = K_ext[0].reshape(16, NR, 64, HQ, DH)
    V = V_ext[0].reshape(16, NR, 64, HQ, DH)

    def body(x_ref, wq_ref, k_ref, v_ref, wo_ref, out_ref,
             xbf, xgat, qscr, kstage, vstage, pacc, pstat,
             rbuf, rstat, xsend, xrecv, asend, arecv, ssend, srecv, cpy,
             kcpy, vcpy):
        my = lax.axis_index("i")

        bar = pltpu.get_barrier_semaphore()
        for j in range(1, N_DEV):
            pl.semaphore_signal(bar, inc=1, device_id=((my + j) % N_DEV,),
                                device_id_type=_MESH)
        pl.semaphore_wait(bar, N_DEV - 1)

        pending = []

        def stage_start(h, slot):
            pltpu.make_async_copy(k_ref.at[:, :, :, h, :], kstage.at[slot],
                                  kcpy.at[slot]).start()
            pltpu.make_async_copy(v_ref.at[:, :, :, h, :], vstage.at[slot],
                                  vcpy.at[slot]).start()

        def stage_wait(h, slot):
            pltpu.make_async_copy(k_ref.at[:, :, :, h, :], kstage.at[slot],
                                  kcpy.at[slot]).wait()
            pltpu.make_async_copy(v_ref.at[:, :, :, h, :], vstage.at[slot],
                                  vcpy.at[slot]).wait()

        stage_start(0, 0)
        stage_start(1, 1)

        xbf[...] = x_ref[...].astype(jnp.bfloat16)
        for j in range(1, N_DEV):
            dst = (my + j) % N_DEV
            rx = pltpu.make_async_remote_copy(
                xbf, xgat.at[my], xsend.at[j - 1], xrecv.at[my],
                device_id=(dst,), device_id_type=_MESH)
            rx.start()
            pending.append(rx)
        cx = pltpu.make_async_copy(xbf, xgat.at[my], cpy.at[0])
        cx.start()
        cx.wait()
        for j in range(1, N_DEV):
            src = (my + j) % N_DEV
            pltpu.make_async_remote_copy(
                xbf, xgat.at[src], xsend.at[j - 1], xrecv.at[src],
                device_id=(src,), device_id_type=_MESH).wait_recv()

        xall = xgat[...].astype(jnp.float32).reshape(D, D)
        for h in range(HQ):
            qh = lax.dot(xall, wq_ref[:, h * DH:(h + 1) * DH],
                         precision=_PREC,
                         preferred_element_type=jnp.float32)
            q4 = qh.reshape(N_DEV, NR, 64, DH)
            for r in range(NR):
                qscr[h, r] = q4[:, r].reshape(SQ, DH)

        def head_body(h, carry):
            slot = h % 3
            stage_wait(h, slot)
            stage_start(jnp.minimum(h + 2, HQ - 1), (h + 2) % 3)
            for r in range(NR):
                qr = qscr[h, r]
                kt = kstage[slot, :, r].reshape(SKR, DH)
                vt = vstage[slot, :, r].reshape(SKR, DH)
                s = lax.dot_general(
                    qr, kt, (((1,), (1,)), ((), ())),
                    precision=_PREC,
                    preferred_element_type=jnp.float32) * SCALE
                m_r = jnp.max(s, axis=1, keepdims=True)
                p = jnp.exp(s - m_r)
                l_r = jnp.sum(p, axis=1, keepdims=True)
                a_r = lax.dot(p, vt,
                              precision=_PREC,
                              preferred_element_type=jnp.float32)
                pacc[:, h, r] = a_r.astype(jnp.bfloat16).reshape(
                    N_DEV, 64, DH)
                pstat[:, h, r] = jnp.concatenate(
                    [m_r, l_r], axis=1).reshape(N_DEV, 64, 2)
            return carry

        HH = HQ // 2

        def half_send(half):
            lo = half * HH
            for j in range(1, N_DEV):
                dst = (my + j) % N_DEV
                ra = pltpu.make_async_remote_copy(
                    pacc.at[dst].at[pl.ds(lo, HH)],
                    rbuf.at[my].at[pl.ds(lo, HH)],
                    asend.at[half].at[j - 1], arecv.at[half].at[my],
                    device_id=(dst,), device_id_type=_MESH)
                rs = pltpu.make_async_remote_copy(
                    pstat.at[dst].at[pl.ds(lo, HH)],
                    rstat.at[my].at[pl.ds(lo, HH)],
                    ssend.at[half].at[j - 1], srecv.at[half].at[my],
                    device_id=(dst,), device_id_type=_MESH)
                ra.start()
                rs.start()
                pending.append(ra)
                pending.append(rs)
            ca = pltpu.make_async_copy(
                pacc.at[my].at[pl.ds(lo, HH)],
                rbuf.at[my].at[pl.ds(lo, HH)], cpy.at[2 * half + 1])
            cs = pltpu.make_async_copy(
                pstat.at[my].at[pl.ds(lo, HH)],
                rstat.at[my].at[pl.ds(lo, HH)], cpy.at[2 * half + 2])
            ca.start()
            cs.start()
            return ca, cs

        def half_wait(half, own):
            own[0].wait()
            own[1].wait()
            lo = half * HH
            for j in range(1, N_DEV):
                src = (my + j) % N_DEV
                pltpu.make_async_remote_copy(
                    pacc.at[src].at[pl.ds(lo, HH)],
                    rbuf.at[src].at[pl.ds(lo, HH)],
                    asend.at[half].at[j - 1], arecv.at[half].at[src],
                    device_id=(src,), device_id_type=_MESH).wait_recv()
                pltpu.make_async_remote_copy(
                    pstat.at[src].at[pl.ds(lo, HH)],
                    rstat.at[src].at[pl.ds(lo, HH)],
                    ssend.at[half].at[j - 1], srecv.at[half].at[src],
                    device_id=(src,), device_id_type=_MESH).wait_recv()

        def combine_half(half):
            lo = half * HH
            a = rbuf[:, lo:lo + HH].astype(jnp.float32).reshape(
                N_DEV, HH * SQ, DH)
            st = rstat[:, lo:lo + HH].reshape(N_DEV, HH * SQ, 2)
            m = st[:, :, 0:1]
            l = st[:, :, 1:2]
            mg = jnp.max(m, axis=0, keepdims=True)
            w = jnp.exp(m - mg)
            num = jnp.sum(w * a, axis=0)
            den = jnp.sum(w * l, axis=0)
            ctx = (num / den).reshape(HH, SQ, DH).transpose(1, 0, 2
                                                           ).reshape(SQ,
                                                                     HH * DH)
            wo_h = wo_ref[lo * DH:(lo + HH) * DH, :]
            return lax.dot(ctx, wo_h, precision=_PREC,
                           preferred_element_type=jnp.float32)

        lax.fori_loop(0, HH, head_body, 0)
        own0 = half_send(0)
        lax.fori_loop(HH, HQ, head_body, 0)
        stage_wait(HQ - 1, 2)
        stage_wait(HQ - 1, 0)
        own1 = half_send(1)
        half_wait(0, own0)
        out0 = combine_half(0)
        half_wait(1, own1)
        out_ref[...] = out0 + combine_half(1)

        for d in pending:
            d.wait_send()

        @functools.partial(pl.run_scoped, sem2=pltpu.SemaphoreType.REGULAR)
        def _(sem2):
            for j in range(1, N_DEV):
                pl.semaphore_signal(sem2, inc=1,
                                    device_id=((my + j) % N_DEV,),
                                    device_id_type=_MESH)
            pl.semaphore_wait(sem2, N_DEV - 1)

    out = pl.pallas_call(
        body,
        out_shape=jax.ShapeDtypeStruct((SQ, D), jnp.float32),
        in_specs=[
            pl.BlockSpec(memory_space=pltpu.MemorySpace.VMEM),
            pl.BlockSpec(memory_space=pltpu.MemorySpace.VMEM),
            pl.BlockSpec(memory_space=pltpu.MemorySpace.HBM),
            pl.BlockSpec(memory_space=pltpu.MemorySpace.HBM),
            pl.BlockSpec(memory_space=pltpu.MemorySpace.VMEM),
        ],
        out_specs=pl.BlockSpec(memory_space=pltpu.MemorySpace.VMEM),
        scratch_shapes=[
            pltpu.VMEM((SQ, D), jnp.bfloat16),
            pltpu.VMEM((N_DEV, SQ, D), jnp.bfloat16),
            pltpu.VMEM((HQ, NR, SQ, DH), jnp.float32),
            pltpu.VMEM((3, 16, NR, 64, DH), jnp.float32),
            pltpu.VMEM((3, 16, NR, 64, DH), jnp.float32),
            pltpu.VMEM((N_DEV, HQ, NR, 64, DH), jnp.bfloat16),
            pltpu.VMEM((N_DEV, HQ, NR, 64, 2), jnp.float32),
            pltpu.VMEM((N_DEV, HQ, NR, 64, DH), jnp.bfloat16),
            pltpu.VMEM((N_DEV, HQ, NR, 64, 2), jnp.float32),
            pltpu.SemaphoreType.DMA((N_DEV - 1,)),
            pltpu.SemaphoreType.DMA((N_DEV,)),
            pltpu.SemaphoreType.DMA((2, N_DEV - 1)),
            pltpu.SemaphoreType.DMA((2, N_DEV)),
            pltpu.SemaphoreType.DMA((2, N_DEV - 1)),
            pltpu.SemaphoreType.DMA((2, N_DEV)),
            pltpu.SemaphoreType.DMA((5,)),
            pltpu.SemaphoreType.DMA((3,)),
            pltpu.SemaphoreType.DMA((3,)),
        ],
        compiler_params=pltpu.CompilerParams(
            collective_id=0, vmem_limit_bytes=52 * 1024 * 1024),
    )(x2, Wq, K, V, Wo)
    return out[None]
